# baseline (speedup 1.0000x reference)
import functools

import jax
import jax.numpy as jnp
from jax.experimental import pallas as pl
from jax.experimental.pallas import tpu as pltpu
from jax.experimental.pallas import tpu_sc as plsc


def _proj_ln_body(masks_ref, w_ref, bias_ref, gamma_ref, beta_ref, q_ref):
    h = jax.lax.dot_general(masks_ref[...], w_ref[...],
                            dimension_numbers=(((1,), (1,)), ((), ())),
                            preferred_element_type=jnp.float32)
    h = h + bias_ref[...]
    mean = jnp.mean(h, axis=1, keepdims=True)
    var = jnp.mean((h - mean) ** 2, axis=1, keepdims=True)
    q = (h - mean) * jax.lax.rsqrt(var + 1e-5) * gamma_ref[...] \
        + beta_ref[...]
    half = q.shape[1] // 2
    lo_bits = jax.lax.bitcast_convert_type(q[:, :half], jnp.uint32)
    hi_bits = jax.lax.bitcast_convert_type(q[:, half:], jnp.uint32)
    lo16 = (lo_bits + 0x8000) >> 16
    hi16 = (hi_bits + 0x8000) & jnp.uint32(0xFFFF0000)
    q_ref[...] = hi16 | lo16


def _compute_table(masks, w, bias, gamma, beta, blk):
    v, bloom = masks.shape
    out = w.shape[0]
    grid = v // blk
    return pl.pallas_call(
        _proj_ln_body,
        grid=(grid,),
        in_specs=[
            pl.BlockSpec((blk, bloom), lambda i: (i, 0)),
            pl.BlockSpec((out, bloom), lambda i: (0, 0)),
            pl.BlockSpec((1, out), lambda i: (0, 0)),
            pl.BlockSpec((1, out), lambda i: (0, 0)),
            pl.BlockSpec((1, out), lambda i: (0, 0)),
        ],
        out_specs=pl.BlockSpec((blk, out // 2), lambda i: (i, 0)),
        out_shape=jax.ShapeDtypeStruct((v, out // 2), jnp.uint32),
        compiler_params=pltpu.CompilerParams(
            dimension_semantics=("parallel",)),
    )(masks, w, bias, gamma, beta)


def _gather_rows(q, idx_flat, win=128):
    ntok = idx_flat.shape[0]
    out = q.shape[1]
    idx2d = idx_flat.reshape(1, ntok)
    mesh = plsc.VectorSubcoreMesh(core_axis_name="c", subcore_axis_name="s")

    @functools.partial(
        pl.kernel,
        out_type=jax.ShapeDtypeStruct((ntok, out), q.dtype),
        mesh=mesh,
    )
    def gather_kernel(q_hbm, i_hbm, o_hbm):
        def body(i_vmem, o_vmem):
            pltpu.sync_copy(q_hbm.at[i_vmem.at[0]], o_vmem)

        pltpu.emit_pipeline(
            body,
            grid=(ntok // win,),
            in_specs=[pl.BlockSpec((1, win), lambda i: (0, i))],
            out_specs=[pl.BlockSpec((win, out), lambda i: (i, 0))],
            core_axis_name=("c", "s"),
            dimension_semantics=(pltpu.PARALLEL,),
        )(i_hbm, o_hbm)

    return gather_kernel(q, idx2d)


def _unpack_body(g_ref, o_ref):
    g = g_ref[...]
    o_ref[:, : g.shape[1]] = jax.lax.bitcast_convert_type(
        g << 16, jnp.float32)
    o_ref[:, g.shape[1]:] = jax.lax.bitcast_convert_type(
        g & jnp.uint32(0xFFFF0000), jnp.float32)


def _unpack_into(gathered, buf, base, total, blk):
    n, half = gathered.shape
    base_blk = base // blk
    out_spec = pl.BlockSpec((blk, 2 * half),
                            lambda i, bb=base_blk: (i + bb, 0))
    out_shape = jax.ShapeDtypeStruct((total, 2 * half), jnp.float32)
    params = pltpu.CompilerParams(dimension_semantics=("parallel",))
    if buf is None:
        return pl.pallas_call(
            _unpack_body,
            grid=(n // blk,),
            in_specs=[pl.BlockSpec((blk, half), lambda i: (i, 0))],
            out_specs=out_spec,
            out_shape=out_shape,
            compiler_params=params,
        )(gathered)

    def body(g_ref, buf_ref, o_ref):
        _unpack_body(g_ref, o_ref)

    return pl.pallas_call(
        body,
        grid=(n // blk,),
        in_specs=[pl.BlockSpec((blk, half), lambda i: (i, 0)),
                  pl.BlockSpec(memory_space=pl.ANY)],
        out_specs=out_spec,
        out_shape=out_shape,
        input_output_aliases={1: 0},
        compiler_params=params,
    )(gathered, buf)


def kernel(input_ids, token_bloom_masks, W, bias, gamma, beta):
    b, s = input_ids.shape
    out = W.shape[0]
    q = _compute_table(token_bloom_masks, W,
                       bias.reshape(1, out), gamma.reshape(1, out),
                       beta.reshape(1, out), blk=4000)
    idx_flat = input_ids.T.reshape(b * s).astype(jnp.int32)
    nchunks = 4
    csz = (b * s) // nchunks
    full = None
    for c in range(nchunks):
        idx_c = jax.lax.slice(idx_flat, (c * csz,), ((c + 1) * csz,))
        g_c = _gather_rows(q, idx_c)
        full = _unpack_into(g_c, full, base=c * csz, total=b * s, blk=6400)
    return full.reshape(s, b, out).transpose(1, 0, 2)


# speedup vs baseline: 1.0360x; 1.0360x over previous
import functools

import jax
import jax.numpy as jnp
from jax.experimental import pallas as pl
from jax.experimental.pallas import tpu as pltpu
from jax.experimental.pallas import tpu_sc as plsc


def _proj_ln_body(masks_ref, w_ref, bias_ref, gamma_ref, beta_ref, q_ref):
    h = jax.lax.dot_general(masks_ref[...], w_ref[...],
                            dimension_numbers=(((1,), (1,)), ((), ())),
                            preferred_element_type=jnp.float32)
    h = h + bias_ref[...]
    mean = jnp.mean(h, axis=1, keepdims=True)
    var = jnp.mean((h - mean) ** 2, axis=1, keepdims=True)
    q = (h - mean) * jax.lax.rsqrt(var + 1e-5) * gamma_ref[...] \
        + beta_ref[...]
    half = q.shape[1] // 2
    lo_bits = jax.lax.bitcast_convert_type(q[:, :half], jnp.uint32)
    hi_bits = jax.lax.bitcast_convert_type(q[:, half:], jnp.uint32)
    lo16 = (lo_bits + 0x8000) >> 16
    hi16 = (hi_bits + 0x8000) & jnp.uint32(0xFFFF0000)
    q_ref[...] = hi16 | lo16


def _compute_table(masks, w, bias, gamma, beta, blk):
    v, bloom = masks.shape
    out = w.shape[0]
    grid = v // blk
    return pl.pallas_call(
        _proj_ln_body,
        grid=(grid,),
        in_specs=[
            pl.BlockSpec((blk, bloom), lambda i: (i, 0)),
            pl.BlockSpec((out, bloom), lambda i: (0, 0)),
            pl.BlockSpec((1, out), lambda i: (0, 0)),
            pl.BlockSpec((1, out), lambda i: (0, 0)),
            pl.BlockSpec((1, out), lambda i: (0, 0)),
        ],
        out_specs=pl.BlockSpec((blk, out // 2), lambda i: (i, 0)),
        out_shape=jax.ShapeDtypeStruct((v, out // 2), jnp.uint32),
        compiler_params=pltpu.CompilerParams(
            dimension_semantics=("parallel",)),
    )(masks, w, bias, gamma, beta)


def _gather_rows(q, idx_flat, win=128):
    ntok = idx_flat.shape[0]
    out = q.shape[1]
    idx2d = idx_flat.reshape(1, ntok)
    mesh = plsc.VectorSubcoreMesh(core_axis_name="c", subcore_axis_name="s")

    @functools.partial(
        pl.kernel,
        out_type=jax.ShapeDtypeStruct((ntok, out), q.dtype),
        mesh=mesh,
    )
    def gather_kernel(q_hbm, i_hbm, o_hbm):
        def body(i_vmem, o_vmem):
            pltpu.sync_copy(q_hbm.at[i_vmem.at[0]], o_vmem)

        pltpu.emit_pipeline(
            body,
            grid=(ntok // win,),
            in_specs=[pl.BlockSpec((1, win), lambda i: (0, i))],
            out_specs=[pl.BlockSpec((win, out), lambda i: (i, 0))],
            core_axis_name=("c", "s"),
            dimension_semantics=(pltpu.PARALLEL,),
        )(i_hbm, o_hbm)

    return gather_kernel(q, idx2d)


def _unpack_body(g_ref, o_ref):
    g = g_ref[...]
    o_ref[:, : g.shape[1]] = jax.lax.bitcast_convert_type(
        g << 16, jnp.float32)
    o_ref[:, g.shape[1]:] = jax.lax.bitcast_convert_type(
        g & jnp.uint32(0xFFFF0000), jnp.float32)


def _unpack_into(gathered, buf, base, total, blk):
    n, half = gathered.shape
    base_blk = base // blk
    out_spec = pl.BlockSpec((blk, 2 * half),
                            lambda i, bb=base_blk: (i + bb, 0))
    out_shape = jax.ShapeDtypeStruct((total, 2 * half), jnp.float32)
    params = pltpu.CompilerParams(dimension_semantics=("parallel",))
    if buf is None:
        return pl.pallas_call(
            _unpack_body,
            grid=(n // blk,),
            in_specs=[pl.BlockSpec((blk, half), lambda i: (i, 0))],
            out_specs=out_spec,
            out_shape=out_shape,
            compiler_params=params,
        )(gathered)

    def body(g_ref, buf_ref, o_ref):
        _unpack_body(g_ref, o_ref)

    return pl.pallas_call(
        body,
        grid=(n // blk,),
        in_specs=[pl.BlockSpec((blk, half), lambda i: (i, 0)),
                  pl.BlockSpec(memory_space=pl.ANY)],
        out_specs=out_spec,
        out_shape=out_shape,
        input_output_aliases={1: 0},
        compiler_params=params,
    )(gathered, buf)


def kernel(input_ids, token_bloom_masks, W, bias, gamma, beta):
    b, s = input_ids.shape
    out = W.shape[0]
    q = _compute_table(token_bloom_masks, W,
                       bias.reshape(1, out), gamma.reshape(1, out),
                       beta.reshape(1, out), blk=4000)
    idx_flat = input_ids.T.reshape(b * s).astype(jnp.int32)
    nchunks = 2
    csz = (b * s) // nchunks
    full = None
    for c in range(nchunks):
        idx_c = jax.lax.slice(idx_flat, (c * csz,), ((c + 1) * csz,))
        g_c = _gather_rows(q, idx_c)
        full = _unpack_into(g_c, full, base=c * csz, total=b * s, blk=6400)
    return full.reshape(s, b, out).transpose(1, 0, 2)


# speedup vs baseline: 1.0598x; 1.0230x over previous
import functools

import jax
import jax.numpy as jnp
from jax.experimental import pallas as pl
from jax.experimental.pallas import tpu as pltpu
from jax.experimental.pallas import tpu_sc as plsc


def _proj_ln_body(masks_ref, w_ref, bias_ref, gamma_ref, beta_ref, q_ref):
    h = jax.lax.dot_general(masks_ref[...], w_ref[...],
                            dimension_numbers=(((1,), (1,)), ((), ())),
                            preferred_element_type=jnp.float32)
    h = h + bias_ref[...]
    mean = jnp.mean(h, axis=1, keepdims=True)
    var = jnp.mean((h - mean) ** 2, axis=1, keepdims=True)
    q = (h - mean) * jax.lax.rsqrt(var + 1e-5) * gamma_ref[...] \
        + beta_ref[...]
    half = q.shape[1] // 2
    lo_bits = jax.lax.bitcast_convert_type(q[:, :half], jnp.uint32)
    hi_bits = jax.lax.bitcast_convert_type(q[:, half:], jnp.uint32)
    lo16 = (lo_bits + 0x8000) >> 16
    hi16 = (hi_bits + 0x8000) & jnp.uint32(0xFFFF0000)
    q_ref[...] = hi16 | lo16


def _compute_table(masks, w, bias, gamma, beta, blk):
    v, bloom = masks.shape
    out = w.shape[0]
    grid = v // blk
    return pl.pallas_call(
        _proj_ln_body,
        grid=(grid,),
        in_specs=[
            pl.BlockSpec((blk, bloom), lambda i: (i, 0)),
            pl.BlockSpec((out, bloom), lambda i: (0, 0)),
            pl.BlockSpec((1, out), lambda i: (0, 0)),
            pl.BlockSpec((1, out), lambda i: (0, 0)),
            pl.BlockSpec((1, out), lambda i: (0, 0)),
        ],
        out_specs=pl.BlockSpec((blk, out // 2), lambda i: (i, 0)),
        out_shape=jax.ShapeDtypeStruct((v, out // 2), jnp.uint32),
        compiler_params=pltpu.CompilerParams(
            dimension_semantics=("parallel",)),
    )(masks, w, bias, gamma, beta)


def _gather_rows(q, idx_flat):
    ntok = idx_flat.shape[0]
    width = q.shape[1]
    nw = 32
    per_w = ntok // nw
    k = 80
    nst = per_w // k
    assert per_w % k == 0 and ntok % nw == 0
    mesh = plsc.VectorSubcoreMesh(core_axis_name="c", subcore_axis_name="s")

    @functools.partial(
        pl.kernel,
        out_type=jax.ShapeDtypeStruct((ntok, width), q.dtype),
        mesh=mesh,
        scratch_types=[
            pltpu.VMEM((per_w,), jnp.int32),
            pltpu.VMEM((k, width), q.dtype),
            pltpu.VMEM((k, width), q.dtype),
            pltpu.SemaphoreType.DMA,
            pltpu.SemaphoreType.DMA,
            pltpu.SemaphoreType.DMA,
            pltpu.SemaphoreType.DMA,
        ],
    )
    def gather_kernel(q_hbm, i_hbm, o_hbm, idx_v, b0, b1, sg0, sg1, so0, so1):
        wid = jax.lax.axis_index("s") * 2 + jax.lax.axis_index("c")
        base = wid * per_w
        pltpu.sync_copy(i_hbm.at[pl.ds(base, per_w)], idx_v)
        bufs, sgs, sos = (b0, b1), (sg0, sg1), (so0, so1)
        gh = [None] * nst
        oh = [None] * nst
        gh[0] = pltpu.async_copy(q_hbm.at[idx_v.at[pl.ds(0, k)]], b0, sg0)
        for g in range(nst):
            if g + 1 < nst:
                if g >= 1:
                    oh[g - 1].wait()
                gh[g + 1] = pltpu.async_copy(
                    q_hbm.at[idx_v.at[pl.ds((g + 1) * k, k)]],
                    bufs[(g + 1) % 2], sgs[(g + 1) % 2])
            gh[g].wait()
            oh[g] = pltpu.async_copy(
                bufs[g % 2], o_hbm.at[pl.ds(base + g * k, k)], sos[g % 2])
        if nst > 1:
            oh[nst - 2].wait()
        oh[nst - 1].wait()

    return gather_kernel(q, idx_flat)


def _unpack_body(g_ref, o_ref):
    g = g_ref[...]
    o_ref[:, : g.shape[1]] = jax.lax.bitcast_convert_type(
        g << 16, jnp.float32)
    o_ref[:, g.shape[1]:] = jax.lax.bitcast_convert_type(
        g & jnp.uint32(0xFFFF0000), jnp.float32)


def _unpack_into(gathered, buf, base, total, blk):
    n, half = gathered.shape
    base_blk = base // blk
    out_spec = pl.BlockSpec((blk, 2 * half),
                            lambda i, bb=base_blk: (i + bb, 0))
    out_shape = jax.ShapeDtypeStruct((total, 2 * half), jnp.float32)
    params = pltpu.CompilerParams(dimension_semantics=("parallel",))
    if buf is None:
        return pl.pallas_call(
            _unpack_body,
            grid=(n // blk,),
            in_specs=[pl.BlockSpec((blk, half), lambda i: (i, 0))],
            out_specs=out_spec,
            out_shape=out_shape,
            compiler_params=params,
        )(gathered)

    def body(g_ref, buf_ref, o_ref):
        _unpack_body(g_ref, o_ref)

    return pl.pallas_call(
        body,
        grid=(n // blk,),
        in_specs=[pl.BlockSpec((blk, half), lambda i: (i, 0)),
                  pl.BlockSpec(memory_space=pl.ANY)],
        out_specs=out_spec,
        out_shape=out_shape,
        input_output_aliases={1: 0},
        compiler_params=params,
    )(gathered, buf)


def kernel(input_ids, token_bloom_masks, W, bias, gamma, beta):
    b, s = input_ids.shape
    out = W.shape[0]
    q = _compute_table(token_bloom_masks, W,
                       bias.reshape(1, out), gamma.reshape(1, out),
                       beta.reshape(1, out), blk=4000)
    idx_flat = input_ids.T.reshape(b * s).astype(jnp.int32)
    nchunks = 2
    csz = (b * s) // nchunks
    full = None
    for c in range(nchunks):
        idx_c = jax.lax.slice(idx_flat, (c * csz,), ((c + 1) * csz,))
        g_c = _gather_rows(q, idx_c)
        full = _unpack_into(g_c, full, base=c * csz, total=b * s, blk=6400)
    return full.reshape(s, b, out).transpose(1, 0, 2)


# speedup vs baseline: 1.0744x; 1.0138x over previous
import functools

import jax
import jax.numpy as jnp
from jax.experimental import pallas as pl
from jax.experimental.pallas import tpu as pltpu
from jax.experimental.pallas import tpu_sc as plsc


def _proj_ln_body(masks_ref, w_ref, bias_ref, gamma_ref, beta_ref, q_ref):
    h = jax.lax.dot_general(masks_ref[...], w_ref[...],
                            dimension_numbers=(((1,), (1,)), ((), ())),
                            preferred_element_type=jnp.float32)
    h = h + bias_ref[...]
    mean = jnp.mean(h, axis=1, keepdims=True)
    var = jnp.mean((h - mean) ** 2, axis=1, keepdims=True)
    q = (h - mean) * jax.lax.rsqrt(var + 1e-5) * gamma_ref[...] \
        + beta_ref[...]
    half = q.shape[1] // 2
    lo_bits = jax.lax.bitcast_convert_type(q[:, :half], jnp.uint32)
    hi_bits = jax.lax.bitcast_convert_type(q[:, half:], jnp.uint32)
    lo16 = (lo_bits + 0x8000) >> 16
    hi16 = (hi_bits + 0x8000) & jnp.uint32(0xFFFF0000)
    q_ref[...] = hi16 | lo16


def _compute_table(masks, w, bias, gamma, beta, blk):
    v, bloom = masks.shape
    out = w.shape[0]
    grid = v // blk
    return pl.pallas_call(
        _proj_ln_body,
        grid=(grid,),
        in_specs=[
            pl.BlockSpec((blk, bloom), lambda i: (i, 0)),
            pl.BlockSpec((out, bloom), lambda i: (0, 0)),
            pl.BlockSpec((1, out), lambda i: (0, 0)),
            pl.BlockSpec((1, out), lambda i: (0, 0)),
            pl.BlockSpec((1, out), lambda i: (0, 0)),
        ],
        out_specs=pl.BlockSpec((blk, out // 2), lambda i: (i, 0)),
        out_shape=jax.ShapeDtypeStruct((v, out // 2), jnp.uint32),
        compiler_params=pltpu.CompilerParams(
            dimension_semantics=("parallel",)),
    )(masks, w, bias, gamma, beta)


def _gather_rows(q, idx_flat):
    ntok = idx_flat.shape[0]
    width = q.shape[1]
    nw = 32
    per_w = ntok // nw
    k = 80
    nst = per_w // k
    assert per_w % k == 0 and ntok % nw == 0
    mesh = plsc.VectorSubcoreMesh(core_axis_name="c", subcore_axis_name="s")

    @functools.partial(
        pl.kernel,
        out_type=jax.ShapeDtypeStruct((ntok, width), q.dtype),
        mesh=mesh,
        scratch_types=[
            pltpu.VMEM((per_w,), jnp.int32),
            pltpu.VMEM((k, width), q.dtype),
            pltpu.VMEM((k, width), q.dtype),
            pltpu.SemaphoreType.DMA,
            pltpu.SemaphoreType.DMA,
            pltpu.SemaphoreType.DMA,
            pltpu.SemaphoreType.DMA,
        ],
    )
    def gather_kernel(q_hbm, i_hbm, o_hbm, idx_v, b0, b1, sg0, sg1, so0, so1):
        wid = jax.lax.axis_index("s") * 2 + jax.lax.axis_index("c")
        base = wid * per_w
        pltpu.sync_copy(i_hbm.at[pl.ds(base, per_w)], idx_v)
        bufs, sgs, sos = (b0, b1), (sg0, sg1), (so0, so1)
        gh = [None] * nst
        oh = [None] * nst
        gh[0] = pltpu.async_copy(q_hbm.at[idx_v.at[pl.ds(0, k)]], b0, sg0)
        for g in range(nst):
            if g + 1 < nst:
                if g >= 1:
                    oh[g - 1].wait()
                gh[g + 1] = pltpu.async_copy(
                    q_hbm.at[idx_v.at[pl.ds((g + 1) * k, k)]],
                    bufs[(g + 1) % 2], sgs[(g + 1) % 2])
            gh[g].wait()
            oh[g] = pltpu.async_copy(
                bufs[g % 2], o_hbm.at[pl.ds(base + g * k, k)], sos[g % 2])
        if nst > 1:
            oh[nst - 2].wait()
        oh[nst - 1].wait()

    return gather_kernel(q, idx_flat)


def _unpack_body(g_ref, o_ref):
    g = g_ref[...]
    o_ref[:, : g.shape[1]] = jax.lax.bitcast_convert_type(
        g << 16, jnp.float32)
    o_ref[:, g.shape[1]:] = jax.lax.bitcast_convert_type(
        g & jnp.uint32(0xFFFF0000), jnp.float32)


def _unpack_into(gathered, buf, base, total, blk):
    n, half = gathered.shape
    base_blk = base // blk
    out_spec = pl.BlockSpec((blk, 2 * half),
                            lambda i, bb=base_blk: (i + bb, 0))
    out_shape = jax.ShapeDtypeStruct((total, 2 * half), jnp.float32)
    params = pltpu.CompilerParams(dimension_semantics=("parallel",))
    if buf is None:
        return pl.pallas_call(
            _unpack_body,
            grid=(n // blk,),
            in_specs=[pl.BlockSpec((blk, half), lambda i: (i, 0))],
            out_specs=out_spec,
            out_shape=out_shape,
            compiler_params=params,
        )(gathered)

    def body(g_ref, buf_ref, o_ref):
        _unpack_body(g_ref, o_ref)

    return pl.pallas_call(
        body,
        grid=(n // blk,),
        in_specs=[pl.BlockSpec((blk, half), lambda i: (i, 0)),
                  pl.BlockSpec(memory_space=pl.ANY)],
        out_specs=out_spec,
        out_shape=out_shape,
        input_output_aliases={1: 0},
        compiler_params=params,
    )(gathered, buf)


def kernel(input_ids, token_bloom_masks, W, bias, gamma, beta):
    b, s = input_ids.shape
    out = W.shape[0]
    q = _compute_table(token_bloom_masks, W,
                       bias.reshape(1, out), gamma.reshape(1, out),
                       beta.reshape(1, out), blk=5000)
    idx_flat = input_ids.T.reshape(b * s).astype(jnp.int32)
    nchunks = 2
    csz = (b * s) // nchunks
    full = None
    for c in range(nchunks):
        idx_c = jax.lax.slice(idx_flat, (c * csz,), ((c + 1) * csz,))
        g_c = _gather_rows(q, idx_c)
        full = _unpack_into(g_c, full, base=c * csz, total=b * s, blk=6400)
    return full.reshape(s, b, out).transpose(1, 0, 2)


# speedup vs baseline: 1.0823x; 1.0073x over previous
import functools

import jax
import jax.numpy as jnp
from jax.experimental import pallas as pl
from jax.experimental.pallas import tpu as pltpu
from jax.experimental.pallas import tpu_sc as plsc


def _proj_ln_body(masks_ref, w_ref, bias_ref, gamma_ref, beta_ref, q_ref):
    h = jax.lax.dot_general(masks_ref[...], w_ref[...],
                            dimension_numbers=(((1,), (1,)), ((), ())),
                            preferred_element_type=jnp.float32)
    h = h + bias_ref[...]
    mean = jnp.mean(h, axis=1, keepdims=True)
    var = jnp.mean((h - mean) ** 2, axis=1, keepdims=True)
    q = (h - mean) * jax.lax.rsqrt(var + 1e-5) * gamma_ref[...] \
        + beta_ref[...]
    half = q.shape[1] // 2
    lo_bits = jax.lax.bitcast_convert_type(q[:, :half], jnp.uint32)
    hi_bits = jax.lax.bitcast_convert_type(q[:, half:], jnp.uint32)
    lo16 = (lo_bits + 0x8000) >> 16
    hi16 = (hi_bits + 0x8000) & jnp.uint32(0xFFFF0000)
    q_ref[...] = hi16 | lo16


def _compute_table(masks, w, bias, gamma, beta, blk):
    v, bloom = masks.shape
    out = w.shape[0]
    grid = v // blk
    return pl.pallas_call(
        _proj_ln_body,
        grid=(grid,),
        in_specs=[
            pl.BlockSpec((blk, bloom), lambda i: (i, 0)),
            pl.BlockSpec((out, bloom), lambda i: (0, 0)),
            pl.BlockSpec((1, out), lambda i: (0, 0)),
            pl.BlockSpec((1, out), lambda i: (0, 0)),
            pl.BlockSpec((1, out), lambda i: (0, 0)),
        ],
        out_specs=pl.BlockSpec((blk, out // 2), lambda i: (i, 0)),
        out_shape=jax.ShapeDtypeStruct((v, out // 2), jnp.uint32),
        compiler_params=pltpu.CompilerParams(
            dimension_semantics=("parallel",)),
    )(masks, w, bias, gamma, beta)


def _gather_rows(q, idx_flat):
    ntok = idx_flat.shape[0]
    width = q.shape[1]
    nw = 32
    per_w = ntok // nw
    k = 80
    nst = per_w // k
    assert per_w % k == 0 and ntok % nw == 0
    mesh = plsc.VectorSubcoreMesh(core_axis_name="c", subcore_axis_name="s")

    @functools.partial(
        pl.kernel,
        out_type=jax.ShapeDtypeStruct((ntok, width), q.dtype),
        mesh=mesh,
        scratch_types=[
            pltpu.VMEM((per_w,), jnp.int32),
            pltpu.VMEM((k, width), q.dtype),
            pltpu.VMEM((k, width), q.dtype),
            pltpu.SemaphoreType.DMA,
            pltpu.SemaphoreType.DMA,
            pltpu.SemaphoreType.DMA,
            pltpu.SemaphoreType.DMA,
        ],
    )
    def gather_kernel(q_hbm, i_hbm, o_hbm, idx_v, b0, b1, sg0, sg1, so0, so1):
        wid = jax.lax.axis_index("s") * 2 + jax.lax.axis_index("c")
        base = wid * per_w
        pltpu.sync_copy(i_hbm.at[pl.ds(base, per_w)], idx_v)
        bufs, sgs, sos = (b0, b1), (sg0, sg1), (so0, so1)
        gh = [None] * nst
        oh = [None] * nst
        gh[0] = pltpu.async_copy(q_hbm.at[idx_v.at[pl.ds(0, k)]], b0, sg0)
        for g in range(nst):
            if g + 1 < nst:
                if g >= 1:
                    oh[g - 1].wait()
                gh[g + 1] = pltpu.async_copy(
                    q_hbm.at[idx_v.at[pl.ds((g + 1) * k, k)]],
                    bufs[(g + 1) % 2], sgs[(g + 1) % 2])
            gh[g].wait()
            oh[g] = pltpu.async_copy(
                bufs[g % 2], o_hbm.at[pl.ds(base + g * k, k)], sos[g % 2])
        if nst > 1:
            oh[nst - 2].wait()
        oh[nst - 1].wait()

    return gather_kernel(q, idx_flat)


def _unpack_body(g_ref, o_ref):
    g = g_ref[...]
    o_ref[:, : g.shape[1]] = jax.lax.bitcast_convert_type(
        g << 16, jnp.float32)
    o_ref[:, g.shape[1]:] = jax.lax.bitcast_convert_type(
        g & jnp.uint32(0xFFFF0000), jnp.float32)


def _unpack_into(gathered, buf, base, total, blk):
    n, half = gathered.shape
    base_blk = base // blk
    out_spec = pl.BlockSpec((blk, 2 * half),
                            lambda i, bb=base_blk: (i + bb, 0))
    out_shape = jax.ShapeDtypeStruct((total, 2 * half), jnp.float32)
    params = pltpu.CompilerParams(dimension_semantics=("parallel",))
    if buf is None:
        return pl.pallas_call(
            _unpack_body,
            grid=(n // blk,),
            in_specs=[pl.BlockSpec((blk, half), lambda i: (i, 0))],
            out_specs=out_spec,
            out_shape=out_shape,
            compiler_params=params,
        )(gathered)

    def body(g_ref, buf_ref, o_ref):
        _unpack_body(g_ref, o_ref)

    return pl.pallas_call(
        body,
        grid=(n // blk,),
        in_specs=[pl.BlockSpec((blk, half), lambda i: (i, 0)),
                  pl.BlockSpec(memory_space=pl.ANY)],
        out_specs=out_spec,
        out_shape=out_shape,
        input_output_aliases={1: 0},
        compiler_params=params,
    )(gathered, buf)


def kernel(input_ids, token_bloom_masks, W, bias, gamma, beta):
    b, s = input_ids.shape
    out = W.shape[0]
    q = _compute_table(token_bloom_masks, W,
                       bias.reshape(1, out), gamma.reshape(1, out),
                       beta.reshape(1, out), blk=10000)
    idx_flat = input_ids.T.reshape(b * s).astype(jnp.int32)
    nchunks = 2
    csz = (b * s) // nchunks
    full = None
    for c in range(nchunks):
        idx_c = jax.lax.slice(idx_flat, (c * csz,), ((c + 1) * csz,))
        g_c = _gather_rows(q, idx_c)
        full = _unpack_into(g_c, full, base=c * csz, total=b * s, blk=6400)
    return full.reshape(s, b, out).transpose(1, 0, 2)
